# baseline (speedup 1.0000x reference)
import jax
import jax.numpy as jnp
from jax import lax
from jax.experimental import pallas as pl
from jax.experimental.pallas import tpu as pltpu
from jax.experimental.pallas import tpu_sc as plsc

_IOU_T = 0.5
_SCORE_T = 0.5
_MAXDET = 100
_NEG_INF = float("-inf")

_T0 = 0.999676
_CAP = 64
_WN = 640
_NW = 32
_NC = 2
_NS = 16


def _conf_body(x_ref, conf_ref, cid_ref):
    x = x_ref[0]
    m = jnp.max(x, axis=1)
    ii = lax.broadcasted_iota(jnp.int32, x.shape, 1)
    cid = jnp.min(jnp.where(x == m[:, None], ii, jnp.int32(2 ** 30)), axis=1)
    conf_ref[0, 0, :] = m
    cid_ref[0, 0, :] = cid


def _nms_body(byx_ref, conf_ref, cid_ref,
              oy0_ref, ox0_ref, oy1_ref, ox1_ref, oconf_ref, ocid_ref, onv_ref,
              work_ref, cy0_ref, cy1_ref, cx0_ref, cx1_ref, area_ref):
    ry0 = byx_ref[0]
    rx0 = byx_ref[1]
    ry1 = byx_ref[2]
    rx1 = byx_ref[3]
    cy0_ref[:, :] = jnp.minimum(ry0, ry1)
    cy1_ref[:, :] = jnp.maximum(ry0, ry1)
    cx0_ref[:, :] = jnp.minimum(rx0, rx1)
    cx1_ref[:, :] = jnp.maximum(rx0, rx1)
    area_ref[:, :] = (cy1_ref[:, :] - cy0_ref[:, :]) * (cx1_ref[:, :] - cx0_ref[:, :])
    conf2 = conf_ref[:, :]
    work_ref[:, :] = jnp.where(conf2 >= _SCORE_T, conf2, _NEG_INF)

    shape = conf2.shape
    acc_shape = (shape[0], 128)

    def body(i, carry):
        nval, ay0, ax0, ay1, ax1, aconf, acid = carry
        w = work_ref[:, :]
        m = jnp.max(w, axis=1, keepdims=True)
        ii = lax.broadcasted_iota(jnp.int32, shape, 1)
        idx = jnp.min(jnp.where(w == m, ii, jnp.int32(2 ** 30)), axis=1,
                      keepdims=True)
        selm = ii == idx

        def pickf(a):
            return jnp.sum(jnp.where(selm, a, jnp.float32(0.0)), axis=1,
                           keepdims=True)

        by0 = pickf(byx_ref[0])
        bx0 = pickf(byx_ref[1])
        by1 = pickf(byx_ref[2])
        bx1 = pickf(byx_ref[3])
        bconf = pickf(conf2)
        bcid = jnp.sum(jnp.where(selm, cid_ref[:, :], jnp.int32(0)), axis=1,
                       keepdims=True)

        valid = m > _NEG_INF

        ymin1 = jnp.minimum(by0, by1)
        ymax1 = jnp.maximum(by0, by1)
        xmin1 = jnp.minimum(bx0, bx1)
        xmax1 = jnp.maximum(bx0, bx1)
        inter_h = jnp.maximum(0.0, jnp.minimum(ymax1, cy1_ref[:, :])
                              - jnp.maximum(ymin1, cy0_ref[:, :]))
        inter_w = jnp.maximum(0.0, jnp.minimum(xmax1, cx1_ref[:, :])
                              - jnp.maximum(xmin1, cx0_ref[:, :]))
        inter = inter_h * inter_w
        area1 = (ymax1 - ymin1) * (xmax1 - xmin1)
        union = area1 + area_ref[:, :] - inter
        iou = jnp.where(union > 0, inter / union, jnp.float32(0.0))
        suppress = (iou > _IOU_T) & valid
        work_ref[:, :] = jnp.where(suppress | selm, _NEG_INF, w)

        colm = lax.broadcasted_iota(jnp.int32, acc_shape, 1) == i
        nval = nval + valid.astype(jnp.int32)
        ay0 = jnp.where(colm, by0, ay0)
        ax0 = jnp.where(colm, bx0, ax0)
        ay1 = jnp.where(colm, by1, ay1)
        ax1 = jnp.where(colm, bx1, ax1)
        aconf = jnp.where(colm, bconf, aconf)
        acid = jnp.where(colm, bcid, acid)
        return nval, ay0, ax0, ay1, ax1, aconf, acid

    zf = jnp.zeros(acc_shape, jnp.float32)
    zi = jnp.zeros(acc_shape, jnp.int32)
    init = (jnp.zeros((shape[0], 1), jnp.int32), zf, zf, zf, zf, zf, zi)
    nval, ay0, ax0, ay1, ax1, aconf, acid = lax.fori_loop(
        0, _MAXDET, body, init)
    oy0_ref[:, :] = ay0
    ox0_ref[:, :] = ax0
    oy1_ref[:, :] = ay1
    ox1_ref[:, :] = ax1
    oconf_ref[:, :] = aconf
    ocid_ref[:, :] = acid
    onv_ref[:, :] = jnp.broadcast_to(nval, acc_shape)


def _nms_call(byx, conf, cid, interpret=False):
    B = conf.shape[0]
    N = conf.shape[1]
    outs = pl.pallas_call(
        _nms_body,
        out_shape=[jax.ShapeDtypeStruct((B, 128), jnp.float32)] * 5
        + [jax.ShapeDtypeStruct((B, 128), jnp.int32)] * 2,
        scratch_shapes=[pltpu.VMEM((B, N), jnp.float32)] * 6,
        interpret=interpret,
    )(byx, conf, cid)
    return outs


def _compact_body(y0_ref, x0_ref, y1_ref, x1_ref, conf_ref, cid_ref,
                  py0_ref, px0_ref, py1_ref, px1_ref, pconf_ref, pcid_ref,
                  pcnt_ref,
                  y0_w, x0_w, y1_w, x1_w, conf_w, cid_w,
                  idx_b, oy0_b, ox0_b, oy1_b, ox1_b, oconf_b, ocid_b, ocnt_b):
    wid = lax.axis_index("s") * _NC + lax.axis_index("c")
    col = wid * _WN
    sl_w = pl.ds(col, _WN)

    pltpu.sync_copy(conf_ref.at[:, sl_w], conf_w)
    pltpu.sync_copy(y0_ref.at[:, sl_w], y0_w)
    pltpu.sync_copy(x0_ref.at[:, sl_w], x0_w)
    pltpu.sync_copy(y1_ref.at[:, sl_w], y1_w)
    pltpu.sync_copy(x1_ref.at[:, sl_w], x1_w)
    pltpu.sync_copy(cid_ref.at[:, sl_w], cid_w)

    iota16 = lax.iota(jnp.int32, 16)

    for b in range(8):
        bvec = iota16 * 0 + b

        def scan_step(i, cnt):
            v = conf_w[b, pl.ds(i * 16, 16)]
            m = v >= _T0
            mi = m.astype(jnp.int32)
            pos = cnt + plsc.cumsum(mi) - 1
            okm = m & (pos < _CAP)
            plsc.store_scatter(idx_b, [bvec, pos], iota16 + i * 16, mask=okm)
            return cnt + plsc.all_reduce_population_count(m)

        cnt = lax.fori_loop(0, _WN // 16, scan_step,
                            jnp.zeros((16,), jnp.int32))
        ocnt_b[0, b, :] = cnt

        def gather_step(j, _):
            raw = idx_b[b, pl.ds(j * 16, 16)]
            valid = (iota16 + j * 16) < cnt
            idxs = jnp.where(valid, raw, 0)
            sl = pl.ds(j * 16, 16)
            oconf_b[0, b, sl] = jnp.where(
                valid, plsc.load_gather(conf_w, [bvec, idxs]), _NEG_INF)
            oy0_b[0, b, sl] = jnp.where(
                valid, plsc.load_gather(y0_w, [bvec, idxs]), 0.0)
            ox0_b[0, b, sl] = jnp.where(
                valid, plsc.load_gather(x0_w, [bvec, idxs]), 0.0)
            oy1_b[0, b, sl] = jnp.where(
                valid, plsc.load_gather(y1_w, [bvec, idxs]), 0.0)
            ox1_b[0, b, sl] = jnp.where(
                valid, plsc.load_gather(x1_w, [bvec, idxs]), 0.0)
            ocid_b[0, b, sl] = jnp.where(
                valid, plsc.load_gather(cid_w, [bvec, idxs]), jnp.int32(0))
            return 0

        lax.fori_loop(0, _CAP // 16, gather_step, 0)

    sl_o = pl.ds(wid, 1)
    pltpu.sync_copy(oy0_b, py0_ref.at[sl_o])
    pltpu.sync_copy(ox0_b, px0_ref.at[sl_o])
    pltpu.sync_copy(oy1_b, py1_ref.at[sl_o])
    pltpu.sync_copy(ox1_b, px1_ref.at[sl_o])
    pltpu.sync_copy(oconf_b, pconf_ref.at[sl_o])
    pltpu.sync_copy(ocid_b, pcid_ref.at[sl_o])
    pltpu.sync_copy(ocnt_b, pcnt_ref.at[sl_o])


def _compact_call(y0, x0, y1, x1, conf, cid):
    B = conf.shape[0]
    f32 = jnp.float32
    i32 = jnp.int32
    mesh = plsc.VectorSubcoreMesh(core_axis_name="c", subcore_axis_name="s",
                                  num_cores=_NC, num_subcores=_NS)
    out_type = (
        [jax.ShapeDtypeStruct((_NW, B, _CAP), f32)] * 5
        + [jax.ShapeDtypeStruct((_NW, B, _CAP), i32)]
        + [jax.ShapeDtypeStruct((_NW, B, 16), i32)]
    )
    scratch = (
        [pltpu.VMEM((B, _WN), f32)] * 5
        + [pltpu.VMEM((B, _WN), i32)]
        + [pltpu.VMEM((B, _CAP), i32)]
        + [pltpu.VMEM((1, B, _CAP), f32)] * 5
        + [pltpu.VMEM((1, B, _CAP), i32)]
        + [pltpu.VMEM((1, B, 16), i32)]
    )
    fn = pl.kernel(_compact_body, out_type=out_type, mesh=mesh,
                   scratch_types=scratch,
                   compiler_params=pltpu.CompilerParams(
                       needs_layout_passes=False))
    return fn(y0, x0, y1, x1, conf, cid)


def kernel(boxes, classes):
    B, N, _ = classes.shape
    CHUNK = 1000
    G = (B * N) // CHUNK
    cls_r = classes.reshape(G, CHUNK, 80)
    conf3, cid3 = pl.pallas_call(
        _conf_body,
        grid=(G,),
        in_specs=[pl.BlockSpec((1, CHUNK, 80), lambda i: (i, 0, 0))],
        out_specs=[pl.BlockSpec((1, 1, CHUNK), lambda i: (i, 0, 0)),
                   pl.BlockSpec((1, 1, CHUNK), lambda i: (i, 0, 0))],
        out_shape=[jax.ShapeDtypeStruct((G, 1, CHUNK), jnp.float32),
                   jax.ShapeDtypeStruct((G, 1, CHUNK), jnp.int32)],
    )(cls_r)

    NPAD = 20480
    pad = NPAD - N
    conf2 = conf3.reshape(B, N)
    cid2 = cid3.reshape(B, N)
    conf_p = jnp.pad(conf2, ((0, 0), (0, pad)), constant_values=-1.0)
    cid_p = jnp.pad(cid2, ((0, 0), (0, pad)))
    byx = jnp.pad(boxes.transpose(2, 0, 1), ((0, 0), (0, 0), (0, pad)))

    py0, px0, py1, px1, pconf, pcid, pcnt = _compact_call(
        byx[0], byx[1], byx[2], byx[3], conf_p, cid_p)
    PW = _NW * _CAP

    def _pool(a):
        return a.transpose(1, 0, 2).reshape(B, PW)

    pool_byx = jnp.stack([_pool(py0), _pool(px0), _pool(py1), _pool(px1)],
                         axis=0)
    fast = _nms_call(pool_byx, _pool(pconf), _pool(pcid))

    overflow = jnp.any(pcnt[:, :, 0] > _CAP)
    short = jnp.any(fast[6][:, 0] < _MAXDET)

    outs = lax.cond(overflow | short,
                    lambda: tuple(_nms_call(byx, conf_p, cid_p)),
                    lambda: tuple(fast))
    oy0, ox0, oy1, ox1, oconf, ocid, onv = outs

    box_pred = jnp.stack(
        [oy0[:, :_MAXDET], ox0[:, :_MAXDET], oy1[:, :_MAXDET],
         ox1[:, :_MAXDET]], axis=-1)
    conf_pred = oconf[:, :_MAXDET]
    class_ids = ocid[:, :_MAXDET]
    valid_det = onv[:, 0]
    return box_pred, conf_pred, class_ids, valid_det


# speedup vs baseline: 4.4846x; 4.4846x over previous
import jax
import jax.numpy as jnp
from jax import lax
from jax.experimental import pallas as pl
from jax.experimental.pallas import tpu as pltpu
from jax.experimental.pallas import tpu_sc as plsc

_IOU_T = 0.5
_SCORE_T = 0.5
_MAXDET = 100
_NEG_INF = float("-inf")

_T0 = 0.999676
_CAP = 64
_WN = 640
_NW = 32
_NC = 2
_NS = 16


def _conf_body(x_ref, conf_ref, cid_ref):
    x = x_ref[0]
    m = jnp.max(x, axis=0, keepdims=True)
    ii = lax.broadcasted_iota(jnp.int32, x.shape, 0)
    cid = jnp.min(jnp.where(x == m, ii, jnp.int32(2 ** 30)), axis=0,
                  keepdims=True)
    n = x.shape[1]
    padw = conf_ref.shape[2] - n
    conf_ref[0, 0:1, :] = jnp.concatenate(
        [m, jnp.full((1, padw), -1.0, jnp.float32)], axis=1)
    cid_ref[0, 0:1, :] = jnp.concatenate(
        [cid, jnp.zeros((1, padw), jnp.int32)], axis=1)


def _nms_body(byx_ref, conf_ref, cid_ref,
              oy0_ref, ox0_ref, oy1_ref, ox1_ref, oconf_ref, ocid_ref, onv_ref,
              work_ref, ry0_ref, rx0_ref, ry1_ref, rx1_ref,
              cy0_ref, cy1_ref, cx0_ref, cx1_ref, area_ref):
    ry0 = byx_ref[:, 0, :]
    rx0 = byx_ref[:, 1, :]
    ry1 = byx_ref[:, 2, :]
    rx1 = byx_ref[:, 3, :]
    ry0_ref[:, :] = ry0
    rx0_ref[:, :] = rx0
    ry1_ref[:, :] = ry1
    rx1_ref[:, :] = rx1
    cy0_ref[:, :] = jnp.minimum(ry0, ry1)
    cy1_ref[:, :] = jnp.maximum(ry0, ry1)
    cx0_ref[:, :] = jnp.minimum(rx0, rx1)
    cx1_ref[:, :] = jnp.maximum(rx0, rx1)
    area_ref[:, :] = (cy1_ref[:, :] - cy0_ref[:, :]) * (cx1_ref[:, :] - cx0_ref[:, :])
    conf2 = conf_ref[:, :]
    work_ref[:, :] = jnp.where(conf2 >= _SCORE_T, conf2, _NEG_INF)

    shape = conf2.shape
    acc_shape = (shape[0], 128)

    def body(i, carry):
        nval, ay0, ax0, ay1, ax1, aconf, acid = carry
        w = work_ref[:, :]
        m = jnp.max(w, axis=1, keepdims=True)
        ii = lax.broadcasted_iota(jnp.int32, shape, 1)
        idx = jnp.min(jnp.where(w == m, ii, jnp.int32(2 ** 30)), axis=1,
                      keepdims=True)
        selm = ii == idx

        def pickf(a):
            return jnp.sum(jnp.where(selm, a, jnp.float32(0.0)), axis=1,
                           keepdims=True)

        by0 = pickf(ry0_ref[:, :])
        bx0 = pickf(rx0_ref[:, :])
        by1 = pickf(ry1_ref[:, :])
        bx1 = pickf(rx1_ref[:, :])
        bconf = pickf(conf2)
        bcid = jnp.sum(jnp.where(selm, cid_ref[:, :], jnp.int32(0)), axis=1,
                       keepdims=True)

        valid = m > _NEG_INF

        ymin1 = jnp.minimum(by0, by1)
        ymax1 = jnp.maximum(by0, by1)
        xmin1 = jnp.minimum(bx0, bx1)
        xmax1 = jnp.maximum(bx0, bx1)
        inter_h = jnp.maximum(0.0, jnp.minimum(ymax1, cy1_ref[:, :])
                              - jnp.maximum(ymin1, cy0_ref[:, :]))
        inter_w = jnp.maximum(0.0, jnp.minimum(xmax1, cx1_ref[:, :])
                              - jnp.maximum(xmin1, cx0_ref[:, :]))
        inter = inter_h * inter_w
        area1 = (ymax1 - ymin1) * (xmax1 - xmin1)
        union = area1 + area_ref[:, :] - inter
        iou = jnp.where(union > 0, inter / union, jnp.float32(0.0))
        suppress = (iou > _IOU_T) & valid
        work_ref[:, :] = jnp.where(suppress | selm, _NEG_INF, w)

        colm = lax.broadcasted_iota(jnp.int32, acc_shape, 1) == i
        nval = nval + valid.astype(jnp.int32)
        ay0 = jnp.where(colm, by0, ay0)
        ax0 = jnp.where(colm, bx0, ax0)
        ay1 = jnp.where(colm, by1, ay1)
        ax1 = jnp.where(colm, bx1, ax1)
        aconf = jnp.where(colm, bconf, aconf)
        acid = jnp.where(colm, bcid, acid)
        return nval, ay0, ax0, ay1, ax1, aconf, acid

    zf = jnp.zeros(acc_shape, jnp.float32)
    zi = jnp.zeros(acc_shape, jnp.int32)
    init = (jnp.zeros((shape[0], 1), jnp.int32), zf, zf, zf, zf, zf, zi)
    nval, ay0, ax0, ay1, ax1, aconf, acid = lax.fori_loop(
        0, _MAXDET, body, init)
    oy0_ref[:, :] = ay0
    ox0_ref[:, :] = ax0
    oy1_ref[:, :] = ay1
    ox1_ref[:, :] = ax1
    oconf_ref[:, :] = aconf
    ocid_ref[:, :] = acid
    onv_ref[:, :] = jnp.broadcast_to(nval, acc_shape)


def _nms_call(byx, conf, cid, interpret=False):
    B = conf.shape[0]
    N = conf.shape[1]
    outs = pl.pallas_call(
        _nms_body,
        out_shape=[jax.ShapeDtypeStruct((B, 128), jnp.float32)] * 5
        + [jax.ShapeDtypeStruct((B, 128), jnp.int32)] * 2,
        scratch_shapes=[pltpu.VMEM((B, N), jnp.float32)] * 10,
        interpret=interpret,
    )(byx, conf, cid)
    return outs


def _compact_body(byxw_ref, conf_ref, cid_ref,
                  py0_ref, px0_ref, py1_ref, px1_ref, pconf_ref, pcid_ref,
                  pcnt_ref,
                  byx_w, conf_w, cid_w,
                  idx_b, oy0_b, ox0_b, oy1_b, ox1_b, oconf_b, ocid_b, ocnt_b):
    wid = lax.axis_index("s") * _NC + lax.axis_index("c")
    col = wid * _WN
    sl_w = pl.ds(col, _WN)

    pltpu.sync_copy(conf_ref.at[:, sl_w], conf_w)
    pltpu.sync_copy(byxw_ref.at[:, sl_w], byx_w)
    pltpu.sync_copy(cid_ref.at[:, sl_w], cid_w)

    iota16 = lax.iota(jnp.int32, 16)

    for b in range(8):
        bvec = iota16 * 0 + b

        def scan_step(i, cnt):
            v = conf_w[b, pl.ds(i * 16, 16)]
            m = v >= _T0
            mi = m.astype(jnp.int32)
            pos = cnt + plsc.cumsum(mi) - 1
            okm = m & (pos < _CAP)
            plsc.store_scatter(idx_b, [bvec, pos], iota16 + i * 16, mask=okm)
            return cnt + plsc.all_reduce_population_count(m)

        cnt = lax.fori_loop(0, _WN // 16, scan_step,
                            jnp.zeros((16,), jnp.int32))
        ocnt_b[0, b, :] = cnt

        def gather_step(j, _):
            raw = idx_b[b, pl.ds(j * 16, 16)]
            valid = (iota16 + j * 16) < cnt
            idxs = jnp.where(valid, raw, 0)
            sl = pl.ds(j * 16, 16)
            oconf_b[0, b, sl] = jnp.where(
                valid, plsc.load_gather(conf_w, [bvec, idxs]), _NEG_INF)
            oy0_b[0, b, sl] = jnp.where(
                valid, plsc.load_gather(byx_w, [bvec * 4 + 0, idxs]), 0.0)
            ox0_b[0, b, sl] = jnp.where(
                valid, plsc.load_gather(byx_w, [bvec * 4 + 1, idxs]), 0.0)
            oy1_b[0, b, sl] = jnp.where(
                valid, plsc.load_gather(byx_w, [bvec * 4 + 2, idxs]), 0.0)
            ox1_b[0, b, sl] = jnp.where(
                valid, plsc.load_gather(byx_w, [bvec * 4 + 3, idxs]), 0.0)
            ocid_b[0, b, sl] = jnp.where(
                valid, plsc.load_gather(cid_w, [bvec, idxs]), jnp.int32(0))
            return 0

        lax.fori_loop(0, _CAP // 16, gather_step, 0)

    sl_o = pl.ds(wid, 1)
    pltpu.sync_copy(oy0_b, py0_ref.at[sl_o])
    pltpu.sync_copy(ox0_b, px0_ref.at[sl_o])
    pltpu.sync_copy(oy1_b, py1_ref.at[sl_o])
    pltpu.sync_copy(ox1_b, px1_ref.at[sl_o])
    pltpu.sync_copy(oconf_b, pconf_ref.at[sl_o])
    pltpu.sync_copy(ocid_b, pcid_ref.at[sl_o])
    pltpu.sync_copy(ocnt_b, pcnt_ref.at[sl_o])


def _compact_call(byxw, conf, cid):
    B = conf.shape[0]
    f32 = jnp.float32
    i32 = jnp.int32
    mesh = plsc.VectorSubcoreMesh(core_axis_name="c", subcore_axis_name="s",
                                  num_cores=_NC, num_subcores=_NS)
    out_type = (
        [jax.ShapeDtypeStruct((_NW, B, _CAP), f32)] * 5
        + [jax.ShapeDtypeStruct((_NW, B, _CAP), i32)]
        + [jax.ShapeDtypeStruct((_NW, B, 16), i32)]
    )
    scratch = (
        [pltpu.VMEM((4 * B, _WN), f32)]
        + [pltpu.VMEM((B, _WN), f32)]
        + [pltpu.VMEM((B, _WN), i32)]
        + [pltpu.VMEM((B, _CAP), i32)]
        + [pltpu.VMEM((1, B, _CAP), f32)] * 5
        + [pltpu.VMEM((1, B, _CAP), i32)]
        + [pltpu.VMEM((1, B, 16), i32)]
    )
    fn = pl.kernel(_compact_body, out_type=out_type, mesh=mesh,
                   scratch_types=scratch,
                   compiler_params=pltpu.CompilerParams(
                       needs_layout_passes=False))
    return fn(byxw, conf, cid)


def kernel(boxes, classes):
    B, N, _ = classes.shape
    NPAD = 20480
    pad = NPAD - N

    cls_t = classes.transpose(0, 2, 1)
    conf3, cid3 = pl.pallas_call(
        _conf_body,
        grid=(B,),
        in_specs=[pl.BlockSpec((1, 80, N), lambda b: (b, 0, 0))],
        out_specs=[pl.BlockSpec((1, 1, NPAD), lambda b: (b, 0, 0)),
                   pl.BlockSpec((1, 1, NPAD), lambda b: (b, 0, 0))],
        out_shape=[jax.ShapeDtypeStruct((B, 1, NPAD), jnp.float32),
                   jax.ShapeDtypeStruct((B, 1, NPAD), jnp.int32)],
    )(cls_t)
    conf_p = conf3.reshape(B, NPAD)
    cid_p = cid3.reshape(B, NPAD)

    byx2 = jnp.pad(boxes.transpose(0, 2, 1), ((0, 0), (0, 0), (0, pad)))
    byxw = byx2.reshape(4 * B, NPAD)

    py0, px0, py1, px1, pconf, pcid, pcnt = _compact_call(byxw, conf_p, cid_p)
    PW = _NW * _CAP

    def _pool(a):
        return a.transpose(1, 0, 2).reshape(B, PW)

    pool_byx = jnp.stack([_pool(py0), _pool(px0), _pool(py1), _pool(px1)],
                         axis=1)
    fast = _nms_call(pool_byx, _pool(pconf), _pool(pcid))

    overflow = jnp.any(pcnt[:, :, 0] > _CAP)
    short = jnp.any(fast[6][:, 0] < _MAXDET)

    outs = lax.cond(overflow | short,
                    lambda: tuple(_nms_call(byx2, conf_p, cid_p)),
                    lambda: tuple(fast))
    oy0, ox0, oy1, ox1, oconf, ocid, onv = outs

    box_pred = jnp.stack(
        [oy0[:, :_MAXDET], ox0[:, :_MAXDET], oy1[:, :_MAXDET],
         ox1[:, :_MAXDET]], axis=-1)
    conf_pred = oconf[:, :_MAXDET]
    class_ids = ocid[:, :_MAXDET]
    valid_det = onv[:, 0]
    return box_pred, conf_pred, class_ids, valid_det


# speedup vs baseline: 4.5957x; 1.0248x over previous
import jax
import jax.numpy as jnp
from jax import lax
from jax.experimental import pallas as pl
from jax.experimental.pallas import tpu as pltpu
from jax.experimental.pallas import tpu_sc as plsc

_IOU_T = 0.5
_SCORE_T = 0.5
_MAXDET = 100
_NEG_INF = float("-inf")

_T0 = 0.999676
_CAP = 48
_WN = 640
_NW = 32
_NC = 2
_NS = 16


def _conf_body(x_ref, conf_ref, cid_ref):
    x = x_ref[0]
    m = jnp.max(x, axis=0, keepdims=True)
    ii = lax.broadcasted_iota(jnp.int32, x.shape, 0)
    cid = jnp.min(jnp.where(x == m, ii, jnp.int32(2 ** 30)), axis=0,
                  keepdims=True)
    n = x.shape[1]
    padw = conf_ref.shape[2] - n
    conf_ref[0, 0:1, :] = jnp.concatenate(
        [m, jnp.full((1, padw), -1.0, jnp.float32)], axis=1)
    cid_ref[0, 0:1, :] = jnp.concatenate(
        [cid, jnp.zeros((1, padw), jnp.int32)], axis=1)


def _nms_body(byx_ref, conf_ref, cid_ref,
              oy0_ref, ox0_ref, oy1_ref, ox1_ref, oconf_ref, ocid_ref, onv_ref,
              work_ref, ry0_ref, rx0_ref, ry1_ref, rx1_ref,
              cy0_ref, cy1_ref, cx0_ref, cx1_ref, area_ref):
    ry0 = byx_ref[:, 0, :]
    rx0 = byx_ref[:, 1, :]
    ry1 = byx_ref[:, 2, :]
    rx1 = byx_ref[:, 3, :]
    ry0_ref[:, :] = ry0
    rx0_ref[:, :] = rx0
    ry1_ref[:, :] = ry1
    rx1_ref[:, :] = rx1
    cy0_ref[:, :] = jnp.minimum(ry0, ry1)
    cy1_ref[:, :] = jnp.maximum(ry0, ry1)
    cx0_ref[:, :] = jnp.minimum(rx0, rx1)
    cx1_ref[:, :] = jnp.maximum(rx0, rx1)
    area_ref[:, :] = (cy1_ref[:, :] - cy0_ref[:, :]) * (cx1_ref[:, :] - cx0_ref[:, :])
    conf2 = conf_ref[:, :]
    work_ref[:, :] = jnp.where(conf2 >= _SCORE_T, conf2, _NEG_INF)

    shape = conf2.shape
    acc_shape = (shape[0], 128)

    def body(i, carry):
        nval, ay0, ax0, ay1, ax1, aconf, acid = carry
        w = work_ref[:, :]
        m = jnp.max(w, axis=1, keepdims=True)
        ii = lax.broadcasted_iota(jnp.int32, shape, 1)
        idx = jnp.min(jnp.where(w == m, ii, jnp.int32(2 ** 30)), axis=1,
                      keepdims=True)
        selm = ii == idx

        def pickf(a):
            return jnp.sum(jnp.where(selm, a, jnp.float32(0.0)), axis=1,
                           keepdims=True)

        by0 = pickf(ry0_ref[:, :])
        bx0 = pickf(rx0_ref[:, :])
        by1 = pickf(ry1_ref[:, :])
        bx1 = pickf(rx1_ref[:, :])
        bconf = pickf(conf2)
        bcid = jnp.sum(jnp.where(selm, cid_ref[:, :], jnp.int32(0)), axis=1,
                       keepdims=True)

        valid = m > _NEG_INF

        ymin1 = jnp.minimum(by0, by1)
        ymax1 = jnp.maximum(by0, by1)
        xmin1 = jnp.minimum(bx0, bx1)
        xmax1 = jnp.maximum(bx0, bx1)
        inter_h = jnp.maximum(0.0, jnp.minimum(ymax1, cy1_ref[:, :])
                              - jnp.maximum(ymin1, cy0_ref[:, :]))
        inter_w = jnp.maximum(0.0, jnp.minimum(xmax1, cx1_ref[:, :])
                              - jnp.maximum(xmin1, cx0_ref[:, :]))
        inter = inter_h * inter_w
        area1 = (ymax1 - ymin1) * (xmax1 - xmin1)
        union = area1 + area_ref[:, :] - inter
        iou = jnp.where(union > 0, inter / union, jnp.float32(0.0))
        suppress = (iou > _IOU_T) & valid
        work_ref[:, :] = jnp.where(suppress | selm, _NEG_INF, w)

        colm = lax.broadcasted_iota(jnp.int32, acc_shape, 1) == i
        nval = nval + valid.astype(jnp.int32)
        ay0 = jnp.where(colm, by0, ay0)
        ax0 = jnp.where(colm, bx0, ax0)
        ay1 = jnp.where(colm, by1, ay1)
        ax1 = jnp.where(colm, bx1, ax1)
        aconf = jnp.where(colm, bconf, aconf)
        acid = jnp.where(colm, bcid, acid)
        return nval, ay0, ax0, ay1, ax1, aconf, acid

    zf = jnp.zeros(acc_shape, jnp.float32)
    zi = jnp.zeros(acc_shape, jnp.int32)
    init = (jnp.zeros((shape[0], 1), jnp.int32), zf, zf, zf, zf, zf, zi)
    nval, ay0, ax0, ay1, ax1, aconf, acid = lax.fori_loop(
        0, _MAXDET, body, init)
    oy0_ref[:, :] = ay0
    ox0_ref[:, :] = ax0
    oy1_ref[:, :] = ay1
    ox1_ref[:, :] = ax1
    oconf_ref[:, :] = aconf
    ocid_ref[:, :] = acid
    onv_ref[:, :] = jnp.broadcast_to(nval, acc_shape)


def _nms_call(byx, conf, cid, interpret=False):
    B = conf.shape[0]
    N = conf.shape[1]
    outs = pl.pallas_call(
        _nms_body,
        out_shape=[jax.ShapeDtypeStruct((B, 128), jnp.float32)] * 5
        + [jax.ShapeDtypeStruct((B, 128), jnp.int32)] * 2,
        scratch_shapes=[pltpu.VMEM((B, N), jnp.float32)] * 10,
        interpret=interpret,
    )(byx, conf, cid)
    return outs


def _compact_body(byxw_ref, conf_ref, cid_ref,
                  py0_ref, px0_ref, py1_ref, px1_ref, pconf_ref, pcid_ref,
                  pcnt_ref,
                  byx_w, conf_w, cid_w,
                  idx_b, oy0_b, ox0_b, oy1_b, ox1_b, oconf_b, ocid_b, ocnt_b):
    wid = lax.axis_index("s") * _NC + lax.axis_index("c")
    col = wid * _WN
    sl_w = pl.ds(col, _WN)

    pltpu.sync_copy(conf_ref.at[:, sl_w], conf_w)
    pltpu.sync_copy(byxw_ref.at[:, sl_w], byx_w)
    pltpu.sync_copy(cid_ref.at[:, sl_w], cid_w)

    iota16 = lax.iota(jnp.int32, 16)

    for b in range(8):
        bvec = iota16 * 0 + b

        def scan_step(i, cnt):
            v = conf_w[b, pl.ds(i * 16, 16)]
            m = v >= _T0
            mi = m.astype(jnp.int32)
            pos = cnt + plsc.cumsum(mi) - 1
            okm = m & (pos < _CAP)
            plsc.store_scatter(idx_b, [bvec, pos], iota16 + i * 16, mask=okm)
            return cnt + plsc.all_reduce_population_count(m)

        cnt = lax.fori_loop(0, _WN // 16, scan_step,
                            jnp.zeros((16,), jnp.int32))
        ocnt_b[0, b, :] = cnt

        def gather_step(j, _):
            raw = idx_b[b, pl.ds(j * 16, 16)]
            valid = (iota16 + j * 16) < cnt
            idxs = jnp.where(valid, raw, 0)
            sl = pl.ds(j * 16, 16)
            oconf_b[0, b, sl] = jnp.where(
                valid, plsc.load_gather(conf_w, [bvec, idxs]), _NEG_INF)
            oy0_b[0, b, sl] = jnp.where(
                valid, plsc.load_gather(byx_w, [bvec * 4 + 0, idxs]), 0.0)
            ox0_b[0, b, sl] = jnp.where(
                valid, plsc.load_gather(byx_w, [bvec * 4 + 1, idxs]), 0.0)
            oy1_b[0, b, sl] = jnp.where(
                valid, plsc.load_gather(byx_w, [bvec * 4 + 2, idxs]), 0.0)
            ox1_b[0, b, sl] = jnp.where(
                valid, plsc.load_gather(byx_w, [bvec * 4 + 3, idxs]), 0.0)
            ocid_b[0, b, sl] = jnp.where(
                valid, plsc.load_gather(cid_w, [bvec, idxs]), jnp.int32(0))
            return 0

        lax.fori_loop(0, _CAP // 16, gather_step, 0)

    sl_o = pl.ds(wid, 1)
    pltpu.sync_copy(oy0_b, py0_ref.at[sl_o])
    pltpu.sync_copy(ox0_b, px0_ref.at[sl_o])
    pltpu.sync_copy(oy1_b, py1_ref.at[sl_o])
    pltpu.sync_copy(ox1_b, px1_ref.at[sl_o])
    pltpu.sync_copy(oconf_b, pconf_ref.at[sl_o])
    pltpu.sync_copy(ocid_b, pcid_ref.at[sl_o])
    pltpu.sync_copy(ocnt_b, pcnt_ref.at[sl_o])


def _compact_call(byxw, conf, cid):
    B = conf.shape[0]
    f32 = jnp.float32
    i32 = jnp.int32
    mesh = plsc.VectorSubcoreMesh(core_axis_name="c", subcore_axis_name="s",
                                  num_cores=_NC, num_subcores=_NS)
    out_type = (
        [jax.ShapeDtypeStruct((_NW, B, _CAP), f32)] * 5
        + [jax.ShapeDtypeStruct((_NW, B, _CAP), i32)]
        + [jax.ShapeDtypeStruct((_NW, B, 16), i32)]
    )
    scratch = (
        [pltpu.VMEM((4 * B, _WN), f32)]
        + [pltpu.VMEM((B, _WN), f32)]
        + [pltpu.VMEM((B, _WN), i32)]
        + [pltpu.VMEM((B, _CAP), i32)]
        + [pltpu.VMEM((1, B, _CAP), f32)] * 5
        + [pltpu.VMEM((1, B, _CAP), i32)]
        + [pltpu.VMEM((1, B, 16), i32)]
    )
    fn = pl.kernel(_compact_body, out_type=out_type, mesh=mesh,
                   scratch_types=scratch,
                   compiler_params=pltpu.CompilerParams(
                       needs_layout_passes=False))
    return fn(byxw, conf, cid)


def kernel(boxes, classes):
    B, N, _ = classes.shape
    NPAD = 20480
    pad = NPAD - N

    cls_t = classes.transpose(0, 2, 1)
    conf3, cid3 = pl.pallas_call(
        _conf_body,
        grid=(B,),
        in_specs=[pl.BlockSpec((1, 80, N), lambda b: (b, 0, 0))],
        out_specs=[pl.BlockSpec((1, 1, NPAD), lambda b: (b, 0, 0)),
                   pl.BlockSpec((1, 1, NPAD), lambda b: (b, 0, 0))],
        out_shape=[jax.ShapeDtypeStruct((B, 1, NPAD), jnp.float32),
                   jax.ShapeDtypeStruct((B, 1, NPAD), jnp.int32)],
    )(cls_t)
    conf_p = conf3.reshape(B, NPAD)
    cid_p = cid3.reshape(B, NPAD)

    byx2 = jnp.pad(boxes.transpose(0, 2, 1), ((0, 0), (0, 0), (0, pad)))
    byxw = byx2.reshape(4 * B, NPAD)

    py0, px0, py1, px1, pconf, pcid, pcnt = _compact_call(byxw, conf_p, cid_p)
    PW = _NW * _CAP

    def _pool(a):
        return a.transpose(1, 0, 2).reshape(B, PW)

    pool_byx = jnp.stack([_pool(py0), _pool(px0), _pool(py1), _pool(px1)],
                         axis=1)
    fast = _nms_call(pool_byx, _pool(pconf), _pool(pcid))

    overflow = jnp.any(pcnt[:, :, 0] > _CAP)
    short = jnp.any(fast[6][:, 0] < _MAXDET)

    outs = lax.cond(overflow | short,
                    lambda: tuple(_nms_call(byx2, conf_p, cid_p)),
                    lambda: tuple(fast))
    oy0, ox0, oy1, ox1, oconf, ocid, onv = outs

    box_pred = jnp.stack(
        [oy0[:, :_MAXDET], ox0[:, :_MAXDET], oy1[:, :_MAXDET],
         ox1[:, :_MAXDET]], axis=-1)
    conf_pred = oconf[:, :_MAXDET]
    class_ids = ocid[:, :_MAXDET]
    valid_det = onv[:, 0]
    return box_pred, conf_pred, class_ids, valid_det


# speedup vs baseline: 4.6129x; 1.0038x over previous
import jax
import jax.numpy as jnp
from jax import lax
from jax.experimental import pallas as pl
from jax.experimental.pallas import tpu as pltpu
from jax.experimental.pallas import tpu_sc as plsc

_IOU_T = 0.5
_SCORE_T = 0.5
_MAXDET = 100
_NEG_INF = float("-inf")

_T0 = 0.999676
_CAP = 48
_WN = 640
_NW = 32
_NC = 2
_NS = 16


def _conf_body(x_ref, conf_ref, cid_ref):
    x = x_ref[0]
    m = jnp.max(x, axis=0, keepdims=True)
    ii = lax.broadcasted_iota(jnp.int32, x.shape, 0)
    cid = jnp.min(jnp.where(x == m, ii, jnp.int32(2 ** 30)), axis=0,
                  keepdims=True)
    n = x.shape[1]
    padw = conf_ref.shape[2] - n
    conf_ref[0, 0:1, :] = jnp.concatenate(
        [m, jnp.full((1, padw), -1.0, jnp.float32)], axis=1)
    cid_ref[0, 0:1, :] = jnp.concatenate(
        [cid, jnp.zeros((1, padw), jnp.int32)], axis=1)


def _nms_body(byx_ref, conf_ref, cid_ref,
              oy0_ref, ox0_ref, oy1_ref, ox1_ref, oconf_ref, ocid_ref, onv_ref,
              work_ref, ry0_ref, rx0_ref, ry1_ref, rx1_ref,
              cy0_ref, cy1_ref, cx0_ref, cx1_ref, area_ref):
    ry0 = byx_ref[:, 0, :]
    rx0 = byx_ref[:, 1, :]
    ry1 = byx_ref[:, 2, :]
    rx1 = byx_ref[:, 3, :]
    ry0_ref[:, :] = ry0
    rx0_ref[:, :] = rx0
    ry1_ref[:, :] = ry1
    rx1_ref[:, :] = rx1
    cy0_ref[:, :] = jnp.minimum(ry0, ry1)
    cy1_ref[:, :] = jnp.maximum(ry0, ry1)
    cx0_ref[:, :] = jnp.minimum(rx0, rx1)
    cx1_ref[:, :] = jnp.maximum(rx0, rx1)
    area_ref[:, :] = (cy1_ref[:, :] - cy0_ref[:, :]) * (cx1_ref[:, :] - cx0_ref[:, :])
    conf2 = conf_ref[:, :]
    work_ref[:, :] = jnp.where(conf2 >= _SCORE_T, conf2, _NEG_INF)

    shape = conf2.shape
    acc_shape = (shape[0], 128)

    def body(i, carry):
        nval, ay0, ax0, ay1, ax1, aconf, acid = carry
        w = work_ref[:, :]
        m = jnp.max(w, axis=1, keepdims=True)
        ii = lax.broadcasted_iota(jnp.int32, shape, 1)
        idx = jnp.min(jnp.where(w == m, ii, jnp.int32(2 ** 30)), axis=1,
                      keepdims=True)
        selm = ii == idx

        def pickf(a):
            return jnp.sum(jnp.where(selm, a, jnp.float32(0.0)), axis=1,
                           keepdims=True)

        by0 = pickf(ry0_ref[:, :])
        bx0 = pickf(rx0_ref[:, :])
        by1 = pickf(ry1_ref[:, :])
        bx1 = pickf(rx1_ref[:, :])
        bconf = pickf(conf2)
        bcid = jnp.sum(jnp.where(selm, cid_ref[:, :], jnp.int32(0)), axis=1,
                       keepdims=True)

        valid = m > _NEG_INF

        ymin1 = jnp.minimum(by0, by1)
        ymax1 = jnp.maximum(by0, by1)
        xmin1 = jnp.minimum(bx0, bx1)
        xmax1 = jnp.maximum(bx0, bx1)
        inter_h = jnp.maximum(0.0, jnp.minimum(ymax1, cy1_ref[:, :])
                              - jnp.maximum(ymin1, cy0_ref[:, :]))
        inter_w = jnp.maximum(0.0, jnp.minimum(xmax1, cx1_ref[:, :])
                              - jnp.maximum(xmin1, cx0_ref[:, :]))
        inter = inter_h * inter_w
        area1 = (ymax1 - ymin1) * (xmax1 - xmin1)
        union = area1 + area_ref[:, :] - inter
        iou = jnp.where(union > 0, inter / union, jnp.float32(0.0))
        suppress = (iou > _IOU_T) & valid
        work_ref[:, :] = jnp.where(suppress | selm, _NEG_INF, w)

        colm = lax.broadcasted_iota(jnp.int32, acc_shape, 1) == i
        nval = nval + valid.astype(jnp.int32)
        ay0 = jnp.where(colm, by0, ay0)
        ax0 = jnp.where(colm, bx0, ax0)
        ay1 = jnp.where(colm, by1, ay1)
        ax1 = jnp.where(colm, bx1, ax1)
        aconf = jnp.where(colm, bconf, aconf)
        acid = jnp.where(colm, bcid, acid)
        return nval, ay0, ax0, ay1, ax1, aconf, acid

    zf = jnp.zeros(acc_shape, jnp.float32)
    zi = jnp.zeros(acc_shape, jnp.int32)
    init = (jnp.zeros((shape[0], 1), jnp.int32), zf, zf, zf, zf, zf, zi)
    nval, ay0, ax0, ay1, ax1, aconf, acid = lax.fori_loop(
        0, _MAXDET, body, init)
    oy0_ref[:, :] = ay0
    ox0_ref[:, :] = ax0
    oy1_ref[:, :] = ay1
    ox1_ref[:, :] = ax1
    oconf_ref[:, :] = aconf
    ocid_ref[:, :] = acid
    onv_ref[:, :] = jnp.broadcast_to(nval, acc_shape)


def _nms_call(byx, conf, cid, interpret=False):
    B = conf.shape[0]
    N = conf.shape[1]
    outs = pl.pallas_call(
        _nms_body,
        out_shape=[jax.ShapeDtypeStruct((B, 128), jnp.float32)] * 5
        + [jax.ShapeDtypeStruct((B, 128), jnp.int32)] * 2,
        scratch_shapes=[pltpu.VMEM((B, N), jnp.float32)] * 10,
        interpret=interpret,
    )(byx, conf, cid)
    return outs


def _nms_fast_body(byx_ref, conf_ref, cid_ref,
                   oy0_ref, ox0_ref, oy1_ref, ox1_ref, oconf_ref, ocid_ref,
                   onv_ref,
                   work_ref, ry0_ref, rx0_ref, ry1_ref, rx1_ref,
                   cy0_ref, cy1_ref, cx0_ref, cx1_ref, area_ref,
                   nv_ref, lv_ref):
    ry0 = byx_ref[:, 0, :]
    rx0 = byx_ref[:, 1, :]
    ry1 = byx_ref[:, 2, :]
    rx1 = byx_ref[:, 3, :]
    ry0_ref[:, :] = ry0
    rx0_ref[:, :] = rx0
    ry1_ref[:, :] = ry1
    rx1_ref[:, :] = rx1
    cy0_ref[:, :] = jnp.minimum(ry0, ry1)
    cy1_ref[:, :] = jnp.maximum(ry0, ry1)
    cx0_ref[:, :] = jnp.minimum(rx0, rx1)
    cx1_ref[:, :] = jnp.maximum(rx0, rx1)
    area_ref[:, :] = (cy1_ref[:, :] - cy0_ref[:, :]) * (cx1_ref[:, :] - cx0_ref[:, :])
    conf2 = conf_ref[:, :]
    work_ref[:, :] = jnp.where(conf2 >= _SCORE_T, conf2, _NEG_INF)

    shape = conf2.shape
    acc_shape = (shape[0], 128)
    BIG = jnp.int32(2 ** 30)
    nv_ref[:, :] = jnp.zeros(acc_shape, jnp.int32)
    lv_ref[:, :] = jnp.ones(acc_shape, jnp.int32)

    def cond(carry):
        return jnp.logical_not(carry[1])

    def body(carry):
        i, _, ay0, ax0, ay1, ax1, aconf, acid = carry
        nv1 = nv_ref[:, 0:1]
        live1 = lv_ref[:, 0:1] > 0
        w = work_ref[:, :]
        ii = lax.broadcasted_iota(jnp.int32, shape, 1)
        m1 = jnp.max(w, axis=1, keepdims=True)
        idx1 = jnp.min(jnp.where(w == m1, ii, BIG), axis=1, keepdims=True)
        sel1 = ii == idx1
        wx = jnp.where(sel1, _NEG_INF, w)
        m2 = jnp.max(wx, axis=1, keepdims=True)
        idx2 = jnp.min(jnp.where(wx == m2, ii, BIG), axis=1, keepdims=True)
        sel2 = ii == idx2

        def pick(selm, a, zero):
            return jnp.sum(jnp.where(selm, a, zero), axis=1, keepdims=True)

        zf = jnp.float32(0.0)
        b1y0 = pick(sel1, ry0_ref[:, :], zf)
        b1x0 = pick(sel1, rx0_ref[:, :], zf)
        b1y1 = pick(sel1, ry1_ref[:, :], zf)
        b1x1 = pick(sel1, rx1_ref[:, :], zf)
        b1c = pick(sel1, conf2, zf)
        b1id = pick(sel1, cid_ref[:, :], jnp.int32(0))
        b2y0 = pick(sel2, ry0_ref[:, :], zf)
        b2x0 = pick(sel2, rx0_ref[:, :], zf)
        b2y1 = pick(sel2, ry1_ref[:, :], zf)
        b2x1 = pick(sel2, rx1_ref[:, :], zf)
        b2c = pick(sel2, conf2, zf)
        b2id = pick(sel2, cid_ref[:, :], jnp.int32(0))
        a2 = pick(sel2, area_ref[:, :], zf)

        ymin1 = jnp.minimum(b1y0, b1y1)
        ymax1 = jnp.maximum(b1y0, b1y1)
        xmin1 = jnp.minimum(b1x0, b1x1)
        xmax1 = jnp.maximum(b1x0, b1x1)
        area1 = (ymax1 - ymin1) * (xmax1 - xmin1)
        ymin2 = jnp.minimum(b2y0, b2y1)
        ymax2 = jnp.maximum(b2y0, b2y1)
        xmin2 = jnp.minimum(b2x0, b2x1)
        xmax2 = jnp.maximum(b2x0, b2x1)
        area2b = (ymax2 - ymin2) * (xmax2 - xmin2)

        ih12 = jnp.maximum(0.0, jnp.minimum(ymax1, ymax2)
                           - jnp.maximum(ymin1, ymin2))
        iw12 = jnp.maximum(0.0, jnp.minimum(xmax1, xmax2)
                           - jnp.maximum(xmin1, xmin2))
        inter12 = ih12 * iw12
        union12 = area1 + a2 - inter12
        iou12 = jnp.where(union12 > 0, inter12 / union12, jnp.float32(0.0))

        commit1 = live1 & (nv1 < _MAXDET) & (m1 > _NEG_INF)
        commit2 = (commit1 & (m2 > _NEG_INF) & (nv1 < _MAXDET - 1)
                   & jnp.logical_not(iou12 > _IOU_T))

        ih1 = jnp.maximum(0.0, jnp.minimum(ymax1, cy1_ref[:, :])
                          - jnp.maximum(ymin1, cy0_ref[:, :]))
        iw1 = jnp.maximum(0.0, jnp.minimum(xmax1, cx1_ref[:, :])
                          - jnp.maximum(xmin1, cx0_ref[:, :]))
        inter1 = ih1 * iw1
        union1 = area1 + area_ref[:, :] - inter1
        iou1 = jnp.where(union1 > 0, inter1 / union1, jnp.float32(0.0))
        ih2 = jnp.maximum(0.0, jnp.minimum(ymax2, cy1_ref[:, :])
                          - jnp.maximum(ymin2, cy0_ref[:, :]))
        iw2 = jnp.maximum(0.0, jnp.minimum(xmax2, cx1_ref[:, :])
                          - jnp.maximum(xmin2, cx0_ref[:, :]))
        inter2 = ih2 * iw2
        union2 = area2b + area_ref[:, :] - inter2
        iou2 = jnp.where(union2 > 0, inter2 / union2, jnp.float32(0.0))

        sup = ((commit1 & (sel1 | (iou1 > _IOU_T)))
               | (commit2 & (sel2 | (iou2 > _IOU_T))))
        work_ref[:, :] = jnp.where(sup, _NEG_INF, w)

        c1i = commit1.astype(jnp.int32)
        c2i = commit2.astype(jnp.int32)
        coli = lax.broadcasted_iota(jnp.int32, acc_shape, 1)
        colm1 = (coli == nv1) & commit1
        colm2 = (coli == nv1 + 1) & commit2
        ay0 = jnp.where(colm2, b2y0, jnp.where(colm1, b1y0, ay0))
        ax0 = jnp.where(colm2, b2x0, jnp.where(colm1, b1x0, ax0))
        ay1 = jnp.where(colm2, b2y1, jnp.where(colm1, b1y1, ay1))
        ax1 = jnp.where(colm2, b2x1, jnp.where(colm1, b1x1, ax1))
        aconf = jnp.where(colm2, b2c, jnp.where(colm1, b1c, aconf))
        acid = jnp.where(colm2, b2id, jnp.where(colm1, b1id, acid))
        nv_new = nv1 + (c1i + c2i)
        lv_new = live1 & (m1 > _NEG_INF)
        nv_ref[:, :] = jnp.broadcast_to(nv_new, acc_shape)
        lv_ref[:, :] = jnp.broadcast_to(lv_new.astype(jnp.int32), acc_shape)
        alldone = ((i + 1) >= _MAXDET) | jnp.all(
            (nv_new >= _MAXDET) | jnp.logical_not(lv_new))
        return (i + 1, alldone, ay0, ax0, ay1, ax1, aconf, acid)

    zf = jnp.zeros(acc_shape, jnp.float32)
    zi = jnp.zeros(acc_shape, jnp.int32)
    init = (jnp.int32(0), jnp.bool_(False), zf, zf, zf, zf, zf, zi)
    out = lax.while_loop(cond, body, init)
    _, _, ay0, ax0, ay1, ax1, aconf, acid = out
    nval = nv_ref[:, 0:1]
    oy0_ref[:, :] = ay0
    ox0_ref[:, :] = ax0
    oy1_ref[:, :] = ay1
    ox1_ref[:, :] = ax1
    oconf_ref[:, :] = aconf
    ocid_ref[:, :] = acid
    onv_ref[:, :] = jnp.broadcast_to(nval, acc_shape)


def _nms_fast_call(byx, conf, cid, interpret=False):
    B = conf.shape[0]
    N = conf.shape[1]
    return pl.pallas_call(
        _nms_fast_body,
        out_shape=[jax.ShapeDtypeStruct((B, 128), jnp.float32)] * 5
        + [jax.ShapeDtypeStruct((B, 128), jnp.int32)] * 2,
        scratch_shapes=[pltpu.VMEM((B, N), jnp.float32)] * 10
        + [pltpu.VMEM((B, 128), jnp.int32)] * 2,
        interpret=interpret,
    )(byx, conf, cid)


def _compact_body(byxw_ref, conf_ref, cid_ref,
                  py0_ref, px0_ref, py1_ref, px1_ref, pconf_ref, pcid_ref,
                  pcnt_ref,
                  byx_w, conf_w, cid_w,
                  idx_b, oy0_b, ox0_b, oy1_b, ox1_b, oconf_b, ocid_b, ocnt_b):
    wid = lax.axis_index("s") * _NC + lax.axis_index("c")
    col = wid * _WN
    sl_w = pl.ds(col, _WN)

    pltpu.sync_copy(conf_ref.at[:, sl_w], conf_w)
    pltpu.sync_copy(byxw_ref.at[:, sl_w], byx_w)
    pltpu.sync_copy(cid_ref.at[:, sl_w], cid_w)

    iota16 = lax.iota(jnp.int32, 16)

    for b in range(8):
        bvec = iota16 * 0 + b

        def scan_step(i, cnt):
            v = conf_w[b, pl.ds(i * 16, 16)]
            m = v >= _T0
            mi = m.astype(jnp.int32)
            pos = cnt + plsc.cumsum(mi) - 1
            okm = m & (pos < _CAP)
            plsc.store_scatter(idx_b, [bvec, pos], iota16 + i * 16, mask=okm)
            return cnt + plsc.all_reduce_population_count(m)

        cnt = lax.fori_loop(0, _WN // 16, scan_step,
                            jnp.zeros((16,), jnp.int32))
        ocnt_b[0, b, :] = cnt

        def gather_step(j, _):
            raw = idx_b[b, pl.ds(j * 16, 16)]
            valid = (iota16 + j * 16) < cnt
            idxs = jnp.where(valid, raw, 0)
            sl = pl.ds(j * 16, 16)
            oconf_b[0, b, sl] = jnp.where(
                valid, plsc.load_gather(conf_w, [bvec, idxs]), _NEG_INF)
            oy0_b[0, b, sl] = jnp.where(
                valid, plsc.load_gather(byx_w, [bvec * 4 + 0, idxs]), 0.0)
            ox0_b[0, b, sl] = jnp.where(
                valid, plsc.load_gather(byx_w, [bvec * 4 + 1, idxs]), 0.0)
            oy1_b[0, b, sl] = jnp.where(
                valid, plsc.load_gather(byx_w, [bvec * 4 + 2, idxs]), 0.0)
            ox1_b[0, b, sl] = jnp.where(
                valid, plsc.load_gather(byx_w, [bvec * 4 + 3, idxs]), 0.0)
            ocid_b[0, b, sl] = jnp.where(
                valid, plsc.load_gather(cid_w, [bvec, idxs]), jnp.int32(0))
            return 0

        lax.fori_loop(0, _CAP // 16, gather_step, 0)

    sl_o = pl.ds(wid, 1)
    pltpu.sync_copy(oy0_b, py0_ref.at[sl_o])
    pltpu.sync_copy(ox0_b, px0_ref.at[sl_o])
    pltpu.sync_copy(oy1_b, py1_ref.at[sl_o])
    pltpu.sync_copy(ox1_b, px1_ref.at[sl_o])
    pltpu.sync_copy(oconf_b, pconf_ref.at[sl_o])
    pltpu.sync_copy(ocid_b, pcid_ref.at[sl_o])
    pltpu.sync_copy(ocnt_b, pcnt_ref.at[sl_o])


def _compact_call(byxw, conf, cid):
    B = conf.shape[0]
    f32 = jnp.float32
    i32 = jnp.int32
    mesh = plsc.VectorSubcoreMesh(core_axis_name="c", subcore_axis_name="s",
                                  num_cores=_NC, num_subcores=_NS)
    out_type = (
        [jax.ShapeDtypeStruct((_NW, B, _CAP), f32)] * 5
        + [jax.ShapeDtypeStruct((_NW, B, _CAP), i32)]
        + [jax.ShapeDtypeStruct((_NW, B, 16), i32)]
    )
    scratch = (
        [pltpu.VMEM((4 * B, _WN), f32)]
        + [pltpu.VMEM((B, _WN), f32)]
        + [pltpu.VMEM((B, _WN), i32)]
        + [pltpu.VMEM((B, _CAP), i32)]
        + [pltpu.VMEM((1, B, _CAP), f32)] * 5
        + [pltpu.VMEM((1, B, _CAP), i32)]
        + [pltpu.VMEM((1, B, 16), i32)]
    )
    fn = pl.kernel(_compact_body, out_type=out_type, mesh=mesh,
                   scratch_types=scratch,
                   compiler_params=pltpu.CompilerParams(
                       needs_layout_passes=False))
    return fn(byxw, conf, cid)


def kernel(boxes, classes):
    B, N, _ = classes.shape
    NPAD = 20480
    pad = NPAD - N

    cls_t = classes.transpose(0, 2, 1)
    conf3, cid3 = pl.pallas_call(
        _conf_body,
        grid=(B,),
        in_specs=[pl.BlockSpec((1, 80, N), lambda b: (b, 0, 0))],
        out_specs=[pl.BlockSpec((1, 1, NPAD), lambda b: (b, 0, 0)),
                   pl.BlockSpec((1, 1, NPAD), lambda b: (b, 0, 0))],
        out_shape=[jax.ShapeDtypeStruct((B, 1, NPAD), jnp.float32),
                   jax.ShapeDtypeStruct((B, 1, NPAD), jnp.int32)],
    )(cls_t)
    conf_p = conf3.reshape(B, NPAD)
    cid_p = cid3.reshape(B, NPAD)

    byx2 = jnp.pad(boxes.transpose(0, 2, 1), ((0, 0), (0, 0), (0, pad)))
    byxw = byx2.reshape(4 * B, NPAD)

    py0, px0, py1, px1, pconf, pcid, pcnt = _compact_call(byxw, conf_p, cid_p)
    PW = _NW * _CAP

    def _pool(a):
        return a.transpose(1, 0, 2).reshape(B, PW)

    pool_byx = jnp.stack([_pool(py0), _pool(px0), _pool(py1), _pool(px1)],
                         axis=1)
    fast = _nms_fast_call(pool_byx, _pool(pconf), _pool(pcid))

    overflow = jnp.any(pcnt[:, :, 0] > _CAP)
    short = jnp.any(fast[6][:, 0] < _MAXDET)

    outs = lax.cond(overflow | short,
                    lambda: tuple(_nms_call(byx2, conf_p, cid_p)),
                    lambda: tuple(fast))
    oy0, ox0, oy1, ox1, oconf, ocid, onv = outs

    box_pred = jnp.stack(
        [oy0[:, :_MAXDET], ox0[:, :_MAXDET], oy1[:, :_MAXDET],
         ox1[:, :_MAXDET]], axis=-1)
    conf_pred = oconf[:, :_MAXDET]
    class_ids = ocid[:, :_MAXDET]
    valid_det = onv[:, 0]
    return box_pred, conf_pred, class_ids, valid_det
